# baseline (speedup 1.0000x reference)
import jax
import jax.numpy as jnp
from jax.experimental import pallas as pl
from jax.experimental.pallas import tpu as pltpu

_D_MODEL = 2048
_N_EXPERTS = 64
_TB = 2048


def _router_body(x_ref, wt_ref, out_ref):
    logits = jnp.dot(x_ref[...], wt_ref[...], preferred_element_type=jnp.float32)
    iota = jax.lax.broadcasted_iota(jnp.int32, logits.shape, 1)

    m1 = jnp.max(logits, axis=1, keepdims=True)
    idx1 = jnp.argmax(logits, axis=1, keepdims=True)
    masked = jnp.where(iota == idx1, -jnp.inf, logits)
    m2 = jnp.max(masked, axis=1, keepdims=True)
    idx2 = jnp.argmax(masked, axis=1, keepdims=True)

    w1 = jax.nn.sigmoid(m1 - m2)
    out_ref[...] = jnp.concatenate(
        [w1, 1.0 - w1, idx1.astype(jnp.float32), idx2.astype(jnp.float32)],
        axis=1,
    )


def kernel(x, W):
    wt = W.T
    n_tokens = x.shape[0]
    grid = (n_tokens // _TB,)
    out = pl.pallas_call(
        _router_body,
        grid=grid,
        in_specs=[
            pl.BlockSpec((_TB, _D_MODEL), lambda i: (i, 0)),
            pl.BlockSpec((_D_MODEL, _N_EXPERTS), lambda i: (0, 0)),
        ],
        out_specs=pl.BlockSpec((_TB, 4), lambda i: (i, 0)),
        out_shape=jax.ShapeDtypeStruct((n_tokens, 4), jnp.float32),
        compiler_params=pltpu.CompilerParams(
            dimension_semantics=("parallel",),
        ),
    )(x, wt)
    return (out[:, 0:2], out[:, 2:4].astype(jnp.int32))


# speedup vs baseline: 1.1417x; 1.1417x over previous
import jax
import jax.numpy as jnp
from jax.experimental import pallas as pl
from jax.experimental.pallas import tpu as pltpu

_D_MODEL = 2048
_N_EXPERTS = 64
_N_TOKENS = 16384
_TB = 2048


def _router_body(x_ref, wt_ref, w_out_ref, e_out_ref):
    logits = jnp.dot(x_ref[...], wt_ref[...], preferred_element_type=jnp.float32)
    iota = jax.lax.broadcasted_iota(jnp.int32, logits.shape, 1)

    m1 = jnp.max(logits, axis=1, keepdims=True)
    idx1 = jnp.argmax(logits, axis=1, keepdims=True)
    masked = jnp.where(iota == idx1, -jnp.inf, logits)
    m2 = jnp.max(masked, axis=1, keepdims=True)
    idx2 = jnp.argmax(masked, axis=1, keepdims=True)

    w1 = jax.nn.sigmoid(m1 - m2)
    w_out_ref[...] = jnp.concatenate([w1, 1.0 - w1], axis=1)
    e_out_ref[...] = jnp.concatenate([idx1, idx2], axis=1)


def kernel(x, W):
    wt = W.T
    n_tokens = x.shape[0]
    grid = (n_tokens // _TB,)
    weights, experts = pl.pallas_call(
        _router_body,
        grid=grid,
        in_specs=[
            pl.BlockSpec((_TB, _D_MODEL), lambda i: (i, 0)),
            pl.BlockSpec((_D_MODEL, _N_EXPERTS), lambda i: (0, 0)),
        ],
        out_specs=[
            pl.BlockSpec((_TB, 2), lambda i: (i, 0)),
            pl.BlockSpec((_TB, 2), lambda i: (i, 0)),
        ],
        out_shape=[
            jax.ShapeDtypeStruct((n_tokens, 2), jnp.float32),
            jax.ShapeDtypeStruct((n_tokens, 2), jnp.int32),
        ],
        compiler_params=pltpu.CompilerParams(
            dimension_semantics=("parallel",),
        ),
    )(x, wt)
    return (weights, experts)
